# Initial kernel scaffold; baseline (speedup 1.0000x reference)
import jax
import jax.numpy as jnp
from jax.experimental import pallas as pl


def kernel(x, W_h, a_h, W_o, a_o):
    raise NotImplementedError("write your pallas kernel here")


# speedup vs baseline: 1.8584x; 1.8584x over previous
import functools

import jax
import jax.numpy as jnp
from jax.experimental import pallas as pl

_ALPHA = 0.2


def _head_attention(xb, W, a, D):
    h = jnp.dot(xb, W, preferred_element_type=jnp.float32)
    f1 = jnp.dot(h, a[:D].reshape(D, 1),
                 preferred_element_type=jnp.float32)
    f2 = jnp.dot(h, a[D:].reshape(D, 1),
                 preferred_element_type=jnp.float32)
    e = f1 + f2.reshape(1, -1)
    e = jnp.maximum(e, _ALPHA * e)
    m = jnp.max(e, axis=-1, keepdims=True)
    p = jnp.exp(e - m)
    s = jnp.sum(p, axis=-1, keepdims=True)
    numer = jnp.dot(p, h, preferred_element_type=jnp.float32)
    return numer / s


def _elu(v):
    return jnp.where(v > 0, v, jnp.exp(jnp.minimum(v, 0.0)) - 1.0)


def _gat_kernel(x_ref, wh_ref, ah_ref, wo_ref, ao_ref, out_ref):
    xb = x_ref[0]
    heads = []
    for i in range(3):
        o = _head_attention(xb, wh_ref[i], ah_ref[i], 2)
        heads.append(_elu(o))
    W2 = wo_ref[0]
    h2 = (jnp.dot(heads[0], W2[0:2], preferred_element_type=jnp.float32)
          + jnp.dot(heads[1], W2[2:4], preferred_element_type=jnp.float32)
          + jnp.dot(heads[2], W2[4:6], preferred_element_type=jnp.float32))
    a2 = ao_ref[0]
    f1 = jnp.dot(h2, a2[:4].reshape(4, 1), preferred_element_type=jnp.float32)
    f2 = jnp.dot(h2, a2[4:].reshape(4, 1), preferred_element_type=jnp.float32)
    e = f1 + f2.reshape(1, -1)
    e = jnp.maximum(e, _ALPHA * e)
    m = jnp.max(e, axis=-1, keepdims=True)
    p = jnp.exp(e - m)
    s = jnp.sum(p, axis=-1, keepdims=True)
    numer = jnp.dot(p, h2, preferred_element_type=jnp.float32)
    out_ref[0] = _elu(numer / s)


@functools.partial(jax.jit, static_argnames=("interpret",))
def kernel(x, W_h, a_h, W_o, a_o, interpret=False):
    B, N, F = x.shape
    out = pl.pallas_call(
        _gat_kernel,
        grid=(B,),
        in_specs=[
            pl.BlockSpec((1, N, F), lambda b: (b, 0, 0)),
            pl.BlockSpec(W_h.shape, lambda b: (0, 0, 0)),
            pl.BlockSpec(a_h.shape, lambda b: (0, 0)),
            pl.BlockSpec(W_o.shape, lambda b: (0, 0, 0)),
            pl.BlockSpec(a_o.shape, lambda b: (0, 0)),
        ],
        out_specs=pl.BlockSpec((1, N, 4), lambda b: (b, 0, 0)),
        out_shape=jax.ShapeDtypeStruct((B, N, 4), jnp.float32),
        interpret=interpret,
    )(x, W_h, a_h, W_o, a_o)
    return out
